# baseline (speedup 1.0000x reference)
import functools

import jax
import jax.numpy as jnp
from jax import lax
from jax.experimental import pallas as pl
from jax.experimental.pallas import tpu as pltpu
from jax.experimental.pallas import tpu_sc as plsc

N = 10000
E = 640000
IN = 26
LM = 512
HID = 128
G4 = 4 * LM
BN = 1000
CH = 128
NCHUNK = E // CH
NWORK = 32
ZBIG = 640
ZSMALL = N - 15 * ZBIG


def _gates(g, c):
    i = jax.nn.sigmoid(g[:, 0:LM])
    f = jax.nn.sigmoid(g[:, LM:2 * LM])
    gg = jnp.tanh(g[:, 2 * LM:3 * LM])
    o = jax.nn.sigmoid(g[:, 3 * LM:4 * LM])
    c = f * c + i * gg
    return o * jnp.tanh(c), c


def _lstm_body(x_ref, wih0_ref, whh0_ref, b0_ref, w2_ref, b1_ref, out_ref,
               a_ref, h1_ref, c1_ref, h2_ref, c2_ref):
    @pl.when(pl.program_id(0) == 0)
    def _():
        for r in (h1_ref, c1_ref, h2_ref, c2_ref):
            r[...] = jnp.zeros_like(r)

    a_ref[...] = jnp.dot(
        x_ref[...], wih0_ref[...], preferred_element_type=jnp.float32
    ) + b0_ref[...]
    bl1 = b1_ref[...]

    g1_0 = a_ref[pl.ds(0, 1), :] + jnp.dot(
        h1_ref[...].astype(jnp.bfloat16), whh0_ref[...],
        preferred_element_type=jnp.float32)

    def step(t, carry):
        h1p, c1, h2, c2, g1 = carry
        h1, c1 = _gates(g1, c1)
        h1b = h1.astype(jnp.bfloat16)
        tn = jnp.where(t + 1 < BN, t + 1, 0)
        g1n = a_ref[pl.ds(tn, 1), :] + jnp.dot(
            h1b, whh0_ref[...], preferred_element_type=jnp.float32)
        hcat = jnp.concatenate([h1b, h2.astype(jnp.bfloat16)], axis=1)
        g2 = bl1 + jnp.dot(hcat, w2_ref[...],
                           preferred_element_type=jnp.float32)
        h2, c2 = _gates(g2, c2)
        out_ref[pl.ds(t, 1), :] = h2
        return (h1, c1, h2, c2, g1n)

    h1, c1, h2, c2, _ = lax.fori_loop(
        0, BN, step,
        (h1_ref[...], c1_ref[...], h2_ref[...], c2_ref[...], g1_0))
    h1_ref[...] = h1
    c1_ref[...] = c1
    h2_ref[...] = h2
    c2_ref[...] = c2


def _lstm2(xp, wih0t, whh0t, b0, w2cat, b1):
    return pl.pallas_call(
        _lstm_body,
        grid=(N // BN,),
        in_specs=[
            pl.BlockSpec((BN, HID), lambda i: (i, 0)),
            pl.BlockSpec((HID, G4), lambda i: (0, 0)),
            pl.BlockSpec((LM, G4), lambda i: (0, 0)),
            pl.BlockSpec((1, G4), lambda i: (0, 0)),
            pl.BlockSpec((2 * LM, G4), lambda i: (0, 0)),
            pl.BlockSpec((1, G4), lambda i: (0, 0)),
        ],
        out_specs=pl.BlockSpec((BN, LM), lambda i: (i, 0)),
        out_shape=jax.ShapeDtypeStruct((N, LM), jnp.float32),
        scratch_shapes=[
            pltpu.VMEM((BN, G4), jnp.float32),
            pltpu.VMEM((1, LM), jnp.float32),
            pltpu.VMEM((1, LM), jnp.float32),
            pltpu.VMEM((1, LM), jnp.float32),
            pltpu.VMEM((1, LM), jnp.float32),
        ],
    )(xp, wih0t, whh0t.astype(jnp.bfloat16), b0.reshape(1, G4),
      w2cat.astype(jnp.bfloat16), b1.reshape(1, G4))


def _embed_body(x_ref, h_ref, awt_ref, lwt_ref, b_ref, out_ref):
    s = (
        jnp.dot(x_ref[...], awt_ref[...], preferred_element_type=jnp.float32)
        + jnp.dot(h_ref[...], lwt_ref[...], preferred_element_type=jnp.float32)
        + b_ref[...]
    )
    out_ref[...] = jnp.maximum(s, 0.0)


def _embed(xp, h2, awt, lwt, b):
    return pl.pallas_call(
        _embed_body,
        grid=(N // BN,),
        in_specs=[
            pl.BlockSpec((BN, HID), lambda i: (i, 0)),
            pl.BlockSpec((BN, LM), lambda i: (i, 0)),
            pl.BlockSpec((HID, LM), lambda i: (0, 0)),
            pl.BlockSpec((LM, LM), lambda i: (0, 0)),
            pl.BlockSpec((1, LM), lambda i: (0, 0)),
        ],
        out_specs=pl.BlockSpec((BN, LM), lambda i: (i, 0)),
        out_shape=jax.ShapeDtypeStruct((N, LM), jnp.float32),
    )(xp, h2, awt, lwt, b.reshape(1, LM))


def _dinv_from(degt_blk):
    deg = degt_blk[:, 0:1] + degt_blk[:, 1:2] + 1.0
    return jnp.where(deg > 0, lax.rsqrt(deg), 0.0)


def _gcn_mm_body(z_ref, wt_ref, degt_ref, xw_ref, y_ref):
    xw = jnp.dot(z_ref[...], wt_ref[...], preferred_element_type=jnp.float32)
    dinv = _dinv_from(degt_ref[...])
    xw_ref[...] = xw
    y_ref[...] = xw * dinv


def _gcn_mm(z, wt, degt):
    k = z.shape[1]
    return pl.pallas_call(
        _gcn_mm_body,
        grid=(N // BN,),
        in_specs=[
            pl.BlockSpec((BN, k), lambda i: (i, 0)),
            pl.BlockSpec((k, HID), lambda i: (0, 0)),
            pl.BlockSpec((BN, 2), lambda i: (i, 0)),
        ],
        out_specs=[
            pl.BlockSpec((BN, HID), lambda i: (i, 0)),
            pl.BlockSpec((BN, HID), lambda i: (i, 0)),
        ],
        out_shape=[
            jax.ShapeDtypeStruct((N, HID), jnp.float32),
            jax.ShapeDtypeStruct((N, HID), jnp.float32),
        ],
    )(z, wt, degt)


def _comb_body(acc_ref, xw_ref, degt_ref, b_ref, out_ref, *, relu):
    dinv = _dinv_from(degt_ref[...])
    s = (acc_ref[0] + acc_ref[1]) * dinv + xw_ref[...] * (dinv * dinv) + b_ref[...]
    out_ref[...] = jnp.maximum(s, 0.0) if relu else s


def _comb(acc, xw, degt, b, relu):
    return pl.pallas_call(
        functools.partial(_comb_body, relu=relu),
        grid=(N // BN,),
        in_specs=[
            pl.BlockSpec((2, BN, HID), lambda i: (0, i, 0)),
            pl.BlockSpec((BN, HID), lambda i: (i, 0)),
            pl.BlockSpec((BN, 2), lambda i: (i, 0)),
            pl.BlockSpec((1, HID), lambda i: (0, 0)),
        ],
        out_specs=pl.BlockSpec((BN, HID), lambda i: (i, 0)),
        out_shape=jax.ShapeDtypeStruct((N, HID), jnp.float32),
    )(acc, xw, degt, b.reshape(1, HID))


def _sc_mesh():
    return plsc.VectorSubcoreMesh(core_axis_name="c", subcore_axis_name="s")


def _sc_deg(col, ew, zeros_n):
    @functools.partial(
        pl.kernel,
        out_type=jax.ShapeDtypeStruct((2, N), jnp.float32),
        mesh=_sc_mesh(),
        scratch_types=[
            pltpu.VMEM((CH,), jnp.int32),
            pltpu.VMEM((CH,), jnp.float32),
            pltpu.VMEM_SHARED((N,), jnp.float32),
        ],
    )
    def k(col_hbm, ew_hbm, zeros_hbm, out_hbm, colv, ewv, acc):
        cid = lax.axis_index("c")
        sid = lax.axis_index("s")
        wid = sid * 2 + cid

        @pl.when(sid == 0)
        def _():
            pltpu.sync_copy(zeros_hbm, acc)
        plsc.subcore_barrier()

        def body(it, carry):
            g = wid + it * NWORK

            @pl.when(g < NCHUNK)
            def _():
                base = pl.multiple_of(g * CH, CH)
                pltpu.sync_copy(col_hbm.at[pl.ds(base, CH)], colv)
                pltpu.sync_copy(ew_hbm.at[pl.ds(base, CH)], ewv)
                pltpu.sync_copy(ewv, acc.at[colv], add=True)
            return carry

        lax.fori_loop(0, (NCHUNK + NWORK - 1) // NWORK, body, 0)
        plsc.subcore_barrier()

        @pl.when(sid == 0)
        def _():
            pltpu.sync_copy(acc, out_hbm.at[cid])

    return k(col, ew, zeros_n)


def _sc_scatter(y, row, col, ew, zeros_nf):
    @functools.partial(
        pl.kernel,
        out_type=jax.ShapeDtypeStruct((2, N, HID), jnp.float32),
        mesh=_sc_mesh(),
        scratch_types=[
            pltpu.VMEM((CH,), jnp.int32),
            pltpu.VMEM((CH,), jnp.int32),
            pltpu.VMEM((CH,), jnp.float32),
            pltpu.VMEM((CH, HID), jnp.float32),
            pltpu.VMEM_SHARED((N, HID), jnp.float32),
            pltpu.SemaphoreType.DMA,
        ],
    )
    def k(y_hbm, row_hbm, col_hbm, ew_hbm, zeros_hbm, out_hbm,
          rowv, colv, ewv, rows, acc, sem):
        cid = lax.axis_index("c")
        sid = lax.axis_index("s")
        wid = sid * 2 + cid
        rbase = pl.multiple_of(sid * ZBIG, 8)

        @pl.when(sid < 15)
        def _():
            pltpu.sync_copy(zeros_hbm.at[pl.ds(rbase, ZBIG)],
                            acc.at[pl.ds(rbase, ZBIG)])

        @pl.when(sid == 15)
        def _():
            pltpu.sync_copy(zeros_hbm.at[pl.ds(15 * ZBIG, ZSMALL)],
                            acc.at[pl.ds(15 * ZBIG, ZSMALL)])
        plsc.subcore_barrier()

        def body(it, carry):
            g = wid + it * NWORK

            @pl.when(g < NCHUNK)
            def _():
                base = pl.multiple_of(g * CH, CH)
                pltpu.sync_copy(row_hbm.at[pl.ds(base, CH)], rowv)
                pltpu.sync_copy(col_hbm.at[pl.ds(base, CH)], colv)
                pltpu.sync_copy(ew_hbm.at[pl.ds(base, CH)], ewv)
                pltpu.async_copy(y_hbm.at[rowv], rows, sem).wait()

                def ebody(e16, c2):
                    wv = ewv[pl.ds(e16 * 16, 16)]
                    for i in range(16):
                        w = wv[i]
                        e = e16 * 16 + i
                        for j in range(HID // 16):
                            sl = pl.ds(j * 16, 16)
                            rows[e, sl] = rows[e, sl] * w
                    return c2

                lax.fori_loop(0, CH // 16, ebody, 0)
                pltpu.sync_copy(rows, acc.at[colv], add=True)
            return carry

        lax.fori_loop(0, (NCHUNK + NWORK - 1) // NWORK, body, 0)
        plsc.subcore_barrier()

        @pl.when(sid < 15)
        def _():
            pltpu.sync_copy(acc.at[pl.ds(rbase, ZBIG)],
                            out_hbm.at[cid, pl.ds(rbase, ZBIG)])

        @pl.when(sid == 15)
        def _():
            pltpu.sync_copy(acc.at[pl.ds(15 * ZBIG, ZSMALL)],
                            out_hbm.at[cid, pl.ds(15 * ZBIG, ZSMALL)])

    return k(y, row, col, ew, zeros_nf)


def kernel(x, edge_index, edge_weight,
           w_ih0, w_hh0, b_ih0, b_hh0,
           w_ih1, w_hh1, b_ih1, b_hh1,
           aa_W, lm_W, lm_b,
           W1, b1, W2, b2, W3, b3):
    xp = jnp.pad(x, ((0, 0), (0, HID - IN)))
    wih0t = jnp.pad(w_ih0.T, ((0, HID - IN), (0, 0)))
    awt = jnp.pad(aa_W.T, ((0, HID - IN), (0, 0)))

    w2cat = jnp.concatenate([w_ih1.T, w_hh1.T], axis=0)
    h2 = _lstm2(xp, wih0t, w_hh0.T, b_ih0 + b_hh0, w2cat, b_ih1 + b_hh1)
    z = _embed(xp, h2, awt, lm_W.T, lm_b)

    row = edge_index[0]
    col = edge_index[1]
    zeros_n = jnp.zeros((N,), jnp.float32)
    zeros_nf = jnp.zeros((N, HID), jnp.float32)

    deg2 = _sc_deg(col, edge_weight, zeros_n)
    degt = deg2.T

    for wmat, bvec, relu in ((W1, b1, True), (W2, b2, True), (W3, b3, False)):
        xw, y = _gcn_mm(z, wmat.T, degt)
        acc = _sc_scatter(y, row, col, edge_weight, zeros_nf)
        z = _comb(acc, xw, degt, bvec, relu)
    return z


# speedup vs baseline: 1.2020x; 1.2020x over previous
import functools

import jax
import jax.numpy as jnp
from jax import lax
from jax.experimental import pallas as pl
from jax.experimental.pallas import tpu as pltpu
from jax.experimental.pallas import tpu_sc as plsc

N = 10000
E = 640000
IN = 26
LM = 512
HID = 128
G4 = 4 * LM
BN = 1000
CH = 128
NCHUNK = E // CH
NWORK = 32
ZBIG = 640
ZSMALL = N - 15 * ZBIG


def _gates(g, c):
    i = jax.nn.sigmoid(g[:, 0:LM])
    f = jax.nn.sigmoid(g[:, LM:2 * LM])
    gg = jnp.tanh(g[:, 2 * LM:3 * LM])
    o = jax.nn.sigmoid(g[:, 3 * LM:4 * LM])
    c = f * c + i * gg
    return o * jnp.tanh(c), c


def _lstm_body(x_ref, wih_ref, whh_ref, b_ref, out_ref, a_ref, h_ref, c_ref):
    @pl.when(pl.program_id(0) == 0)
    def _():
        h_ref[...] = jnp.zeros_like(h_ref)
        c_ref[...] = jnp.zeros_like(c_ref)

    a_ref[...] = jnp.dot(
        x_ref[...], wih_ref[...], preferred_element_type=jnp.float32
    ) + b_ref[...]

    def step(t, carry):
        h, c = carry
        g = a_ref[pl.ds(t, 1), :] + jnp.dot(
            h.astype(jnp.bfloat16), whh_ref[...],
            preferred_element_type=jnp.float32)
        h, c = _gates(g, c)
        out_ref[pl.ds(t, 1), :] = h
        return (h, c)

    h, c = lax.fori_loop(0, BN, step, (h_ref[...], c_ref[...]))
    h_ref[...] = h
    c_ref[...] = c


def _lstm_scan(x, wih_t, whh_t, b):
    k = x.shape[1]
    return pl.pallas_call(
        _lstm_body,
        grid=(N // BN,),
        in_specs=[
            pl.BlockSpec((BN, k), lambda i: (i, 0)),
            pl.BlockSpec((k, G4), lambda i: (0, 0)),
            pl.BlockSpec((LM, G4), lambda i: (0, 0)),
            pl.BlockSpec((1, G4), lambda i: (0, 0)),
        ],
        out_specs=pl.BlockSpec((BN, LM), lambda i: (i, 0)),
        out_shape=jax.ShapeDtypeStruct((N, LM), jnp.float32),
        scratch_shapes=[
            pltpu.VMEM((BN, G4), jnp.float32),
            pltpu.VMEM((1, LM), jnp.float32),
            pltpu.VMEM((1, LM), jnp.float32),
        ],
    )(x, wih_t, whh_t.astype(jnp.bfloat16), b.reshape(1, G4))


def _embed_body(x_ref, h_ref, awt_ref, lwt_ref, b_ref, out_ref):
    s = (
        jnp.dot(x_ref[...], awt_ref[...], preferred_element_type=jnp.float32)
        + jnp.dot(h_ref[...], lwt_ref[...], preferred_element_type=jnp.float32)
        + b_ref[...]
    )
    out_ref[...] = jnp.maximum(s, 0.0)


def _embed(xp, h2, awt, lwt, b):
    return pl.pallas_call(
        _embed_body,
        grid=(N // BN,),
        in_specs=[
            pl.BlockSpec((BN, HID), lambda i: (i, 0)),
            pl.BlockSpec((BN, LM), lambda i: (i, 0)),
            pl.BlockSpec((HID, LM), lambda i: (0, 0)),
            pl.BlockSpec((LM, LM), lambda i: (0, 0)),
            pl.BlockSpec((1, LM), lambda i: (0, 0)),
        ],
        out_specs=pl.BlockSpec((BN, LM), lambda i: (i, 0)),
        out_shape=jax.ShapeDtypeStruct((N, LM), jnp.float32),
    )(xp, h2, awt, lwt, b.reshape(1, LM))


def _dinv_from(degt_blk):
    deg = degt_blk[:, 0:1] + degt_blk[:, 1:2] + 1.0
    return jnp.where(deg > 0, lax.rsqrt(deg), 0.0)


def _gcn_mm_body(z_ref, wt_ref, degt_ref, xw_ref, y_ref):
    xw = jnp.dot(z_ref[...], wt_ref[...], preferred_element_type=jnp.float32)
    dinv = _dinv_from(degt_ref[...])
    xw_ref[...] = xw
    y_ref[...] = xw * dinv


def _gcn_mm(z, wt, degt):
    k = z.shape[1]
    return pl.pallas_call(
        _gcn_mm_body,
        grid=(N // BN,),
        in_specs=[
            pl.BlockSpec((BN, k), lambda i: (i, 0)),
            pl.BlockSpec((k, HID), lambda i: (0, 0)),
            pl.BlockSpec((BN, 2), lambda i: (i, 0)),
        ],
        out_specs=[
            pl.BlockSpec((BN, HID), lambda i: (i, 0)),
            pl.BlockSpec((BN, HID), lambda i: (i, 0)),
        ],
        out_shape=[
            jax.ShapeDtypeStruct((N, HID), jnp.float32),
            jax.ShapeDtypeStruct((N, HID), jnp.float32),
        ],
    )(z, wt, degt)


def _comb_body(acc_ref, xw_ref, degt_ref, b_ref, out_ref, *, relu):
    dinv = _dinv_from(degt_ref[...])
    s = (acc_ref[0] + acc_ref[1]) * dinv + xw_ref[...] * (dinv * dinv) + b_ref[...]
    out_ref[...] = jnp.maximum(s, 0.0) if relu else s


def _comb(acc, xw, degt, b, relu):
    return pl.pallas_call(
        functools.partial(_comb_body, relu=relu),
        grid=(N // BN,),
        in_specs=[
            pl.BlockSpec((2, BN, HID), lambda i: (0, i, 0)),
            pl.BlockSpec((BN, HID), lambda i: (i, 0)),
            pl.BlockSpec((BN, 2), lambda i: (i, 0)),
            pl.BlockSpec((1, HID), lambda i: (0, 0)),
        ],
        out_specs=pl.BlockSpec((BN, HID), lambda i: (i, 0)),
        out_shape=jax.ShapeDtypeStruct((N, HID), jnp.float32),
    )(acc, xw, degt, b.reshape(1, HID))


def _sc_mesh():
    return plsc.VectorSubcoreMesh(core_axis_name="c", subcore_axis_name="s")


def _sc_deg(col, ew, zeros_n):
    @functools.partial(
        pl.kernel,
        out_type=jax.ShapeDtypeStruct((2, N), jnp.float32),
        mesh=_sc_mesh(),
        scratch_types=[
            pltpu.VMEM((CH,), jnp.int32),
            pltpu.VMEM((CH,), jnp.float32),
            pltpu.VMEM_SHARED((N,), jnp.float32),
        ],
    )
    def k(col_hbm, ew_hbm, zeros_hbm, out_hbm, colv, ewv, acc):
        cid = lax.axis_index("c")
        sid = lax.axis_index("s")
        wid = sid * 2 + cid

        @pl.when(sid == 0)
        def _():
            pltpu.sync_copy(zeros_hbm, acc)
        plsc.subcore_barrier()

        def body(it, carry):
            g = wid + it * NWORK

            @pl.when(g < NCHUNK)
            def _():
                base = pl.multiple_of(g * CH, CH)
                pltpu.sync_copy(col_hbm.at[pl.ds(base, CH)], colv)
                pltpu.sync_copy(ew_hbm.at[pl.ds(base, CH)], ewv)
                pltpu.sync_copy(ewv, acc.at[colv], add=True)
            return carry

        lax.fori_loop(0, (NCHUNK + NWORK - 1) // NWORK, body, 0)
        plsc.subcore_barrier()

        @pl.when(sid == 0)
        def _():
            pltpu.sync_copy(acc, out_hbm.at[cid])

    return k(col, ew, zeros_n)


def _sc_scatter(y, row, col, ew, zeros_nf):
    @functools.partial(
        pl.kernel,
        out_type=jax.ShapeDtypeStruct((2, N, HID), jnp.float32),
        mesh=_sc_mesh(),
        scratch_types=[
            pltpu.VMEM((CH,), jnp.int32), pltpu.VMEM((CH,), jnp.int32),
            pltpu.VMEM((CH,), jnp.float32), pltpu.VMEM((CH, HID), jnp.float32),
            pltpu.VMEM((CH,), jnp.int32), pltpu.VMEM((CH,), jnp.int32),
            pltpu.VMEM((CH,), jnp.float32), pltpu.VMEM((CH, HID), jnp.float32),
            pltpu.VMEM_SHARED((N, HID), jnp.float32),
            pltpu.SemaphoreType.DMA, pltpu.SemaphoreType.DMA,
            pltpu.SemaphoreType.DMA, pltpu.SemaphoreType.DMA,
        ],
    )
    def k(y_hbm, row_hbm, col_hbm, ew_hbm, zeros_hbm, out_hbm,
          rowva, colva, ewva, rowsa, rowvb, colvb, ewvb, rowsb, acc,
          isema, isemb, gsema, gsemb):
        cid = lax.axis_index("c")
        sid = lax.axis_index("s")
        wid = sid * 2 + cid
        rbase = pl.multiple_of(sid * ZBIG, 8)

        @pl.when(sid < 15)
        def _():
            pltpu.sync_copy(zeros_hbm.at[pl.ds(rbase, ZBIG)],
                            acc.at[pl.ds(rbase, ZBIG)])

        @pl.when(sid == 15)
        def _():
            pltpu.sync_copy(zeros_hbm.at[pl.ds(15 * ZBIG, ZSMALL)],
                            acc.at[pl.ds(15 * ZBIG, ZSMALL)])
        plsc.subcore_barrier()

        def scale(ewv, rows):
            def ebody(e16, c2):
                wv = ewv[pl.ds(e16 * 16, 16)]
                for i in range(16):
                    w = wv[i]
                    e = e16 * 16 + i
                    for j in range(HID // 16):
                        sl = pl.ds(j * 16, 16)
                        rows[e, sl] = rows[e, sl] * w
                return c2
            lax.fori_loop(0, CH // 16, ebody, 0)

        def idx_copies(g, rowv, colv, ewv, sem):
            base = pl.multiple_of(g * CH, CH)
            return (
                pltpu.make_async_copy(row_hbm.at[pl.ds(base, CH)], rowv, sem),
                pltpu.make_async_copy(col_hbm.at[pl.ds(base, CH)], colv, sem),
                pltpu.make_async_copy(ew_hbm.at[pl.ds(base, CH)], ewv, sem),
            )

        def body(it2, carry):
            ga = wid + (2 * it2) * NWORK
            gb = ga + NWORK

            @pl.when(ga < NCHUNK)
            def _():
                for c in idx_copies(ga, rowva, colva, ewva, isema):
                    c.start()

            @pl.when(gb < NCHUNK)
            def _():
                for c in idx_copies(gb, rowvb, colvb, ewvb, isemb):
                    c.start()

            @pl.when(ga < NCHUNK)
            def _():
                for c in idx_copies(ga, rowva, colva, ewva, isema):
                    c.wait()
                pltpu.async_copy(y_hbm.at[rowva], rowsa, gsema)

            @pl.when(gb < NCHUNK)
            def _():
                for c in idx_copies(gb, rowvb, colvb, ewvb, isemb):
                    c.wait()
                pltpu.async_copy(y_hbm.at[rowvb], rowsb, gsemb)

            @pl.when(ga < NCHUNK)
            def _():
                pltpu.make_async_copy(y_hbm.at[rowva], rowsa, gsema).wait()
                scale(ewva, rowsa)
                pltpu.sync_copy(rowsa, acc.at[colva], add=True)

            @pl.when(gb < NCHUNK)
            def _():
                pltpu.make_async_copy(y_hbm.at[rowvb], rowsb, gsemb).wait()
                scale(ewvb, rowsb)
                pltpu.sync_copy(rowsb, acc.at[colvb], add=True)
            return carry

        lax.fori_loop(0, (NCHUNK + 2 * NWORK - 1) // (2 * NWORK), body, 0)
        plsc.subcore_barrier()

        @pl.when(sid < 15)
        def _():
            pltpu.sync_copy(acc.at[pl.ds(rbase, ZBIG)],
                            out_hbm.at[cid, pl.ds(rbase, ZBIG)])

        @pl.when(sid == 15)
        def _():
            pltpu.sync_copy(acc.at[pl.ds(15 * ZBIG, ZSMALL)],
                            out_hbm.at[cid, pl.ds(15 * ZBIG, ZSMALL)])

    return k(y, row, col, ew, zeros_nf)


def kernel(x, edge_index, edge_weight,
           w_ih0, w_hh0, b_ih0, b_hh0,
           w_ih1, w_hh1, b_ih1, b_hh1,
           aa_W, lm_W, lm_b,
           W1, b1, W2, b2, W3, b3):
    xp = jnp.pad(x, ((0, 0), (0, HID - IN)))
    wih0t = jnp.pad(w_ih0.T, ((0, HID - IN), (0, 0)))
    awt = jnp.pad(aa_W.T, ((0, HID - IN), (0, 0)))

    h1 = _lstm_scan(xp, wih0t, w_hh0.T, b_ih0 + b_hh0)
    h2 = _lstm_scan(h1, w_ih1.T, w_hh1.T, b_ih1 + b_hh1)
    z = _embed(xp, h2, awt, lm_W.T, lm_b)

    row = edge_index[0]
    col = edge_index[1]
    zeros_n = jnp.zeros((N,), jnp.float32)
    zeros_nf = jnp.zeros((N, HID), jnp.float32)

    deg2 = _sc_deg(col, edge_weight, zeros_n)
    degt = deg2.T

    for wmat, bvec, relu in ((W1, b1, True), (W2, b2, True), (W3, b3, False)):
        xw, y = _gcn_mm(z, wmat.T, degt)
        acc = _sc_scatter(y, row, col, edge_weight, zeros_nf)
        z = _comb(acc, xw, degt, bvec, relu)
    return z
